# Initial kernel scaffold; baseline (speedup 1.0000x reference)
import jax
import jax.numpy as jnp
from jax.experimental import pallas as pl


def kernel(qst, data, W_in, b_in, W_out, b_out, W_router, b_router, W_gauss, b_gauss, W1, b1, W2, b2, gamma, beta):
    raise NotImplementedError("write your pallas kernel here")


# speedup vs baseline: 5.2414x; 5.2414x over previous
import functools
import numpy as np

import jax
import jax.numpy as jnp
from jax.experimental import pallas as pl
from jax.experimental.pallas import tpu as pltpu

B, T, C, H, E, K = 8, 512, 768, 12, 8, 2
DH = C // H
TR = T // 8
SIGMA = 9
MARGIN = 1.0 / (2 * E)
NS = 16


def _k1_body(qst_ref, data_ref, Wq_ref, Wk_ref, Wv_ref, bq_ref, bk_ref,
             bv_ref, Wout_ref, bout_ref, Wr_ref, br_ref, Wg_ref, bg_ref,
             eflat_ref, gwbd_ref):
    f32 = jnp.float32
    q = jax.lax.dot_general(qst_ref[...], Wq_ref[...],
                            (((1,), (1,)), ((), ())),
                            preferred_element_type=f32) + bq_ref[...]
    headmask = (jax.lax.broadcasted_iota(jnp.int32, (H, C), 1) // DH
                == jax.lax.broadcasted_iota(jnp.int32, (H, C), 0))
    qbd = (q[:, None, :] * headmask[None].astype(f32)).reshape(B * H, C)
    u_all = jax.lax.dot_general(qbd, Wk_ref[...], (((1,), (0,)), ((), ())),
                                preferred_element_type=f32)
    s0 = jnp.sum(qbd * bk_ref[...], axis=1, keepdims=True)
    inv_sqrt_dh = 1.0 / np.sqrt(DH).astype(np.float32)
    m_rows = []
    for b in range(B):
        ub = u_all[b * H:(b + 1) * H, :]
        db = data_ref[b]
        sc = (jax.lax.dot_general(ub, db, (((1,), (1,)), ((), ())),
                                  preferred_element_type=f32)
              + s0[b * H:(b + 1) * H, :]) * inv_sqrt_dh
        sc = sc - jnp.max(sc, axis=1, keepdims=True)
        esc = jnp.exp(sc)
        attn = esc / jnp.sum(esc, axis=1, keepdims=True)
        m_rows.append(jax.lax.dot_general(attn, db, (((1,), (0,)), ((), ())),
                                          preferred_element_type=f32))
    m_all = jnp.concatenate(m_rows, axis=0)
    f_all = jax.lax.dot_general(m_all, Wv_ref[...], (((1,), (1,)), ((), ())),
                                preferred_element_type=f32)
    f_resh = f_all.reshape(B, H, C)
    ctx = jnp.sum(f_resh * headmask[None].astype(f32), axis=1) + bv_ref[...]
    temp_w = jax.lax.dot_general(ctx, Wout_ref[...], (((1,), (1,)), ((), ())),
                                 preferred_element_type=f32) + bout_ref[...]
    logits = jax.lax.dot_general(temp_w, Wr_ref[...], (((1,), (1,)), ((), ())),
                                 preferred_element_type=f32) + br_ref[...]
    logits = logits - jnp.max(logits, axis=1, keepdims=True)
    el = jnp.exp(logits)
    probs = el / jnp.sum(el, axis=1, keepdims=True)
    iota_e = jax.lax.broadcasted_iota(jnp.int32, (B, E), 1)
    p1 = jnp.max(probs, axis=1, keepdims=True)
    i1 = jnp.min(jnp.where(probs == p1, iota_e, E), axis=1, keepdims=True)
    masked = jnp.where(iota_e == i1, -1.0, probs)
    p2 = jnp.max(masked, axis=1, keepdims=True)
    i2 = jnp.min(jnp.where(masked == p2, iota_e, E), axis=1, keepdims=True)
    psum = p1 + p2
    p1n, p2n = p1 / psum, p2 / psum
    gc = jax.lax.dot_general(temp_w, Wg_ref[...], (((1,), (1,)), ((), ())),
                             preferred_element_type=f32) + bg_ref[...]
    c0 = jnp.tanh(gc[:, :E]) * MARGIN
    c1 = jax.nn.sigmoid(gc[:, E:])
    centers = MARGIN + iota_e.astype(f32) * ((1.0 - 2 * MARGIN) / (E - 1))
    adjusted = centers + c0
    oh1 = (iota_e == i1).astype(f32)
    oh2 = (iota_e == i2).astype(f32)
    c_sel = [jnp.sum(adjusted * oh, axis=1, keepdims=True) for oh in (oh1, oh2)]
    w_sel = [jnp.sum(c1 * oh, axis=1, keepdims=True) for oh in (oh1, oh2)]
    pk = [p1n, p2n]
    eflat_ref[...] = jnp.concatenate([i1, i2], axis=1).T
    iota_l = jax.lax.broadcasted_iota(jnp.int32, (B, B * TR), 1)
    jcol = (iota_l % TR).astype(f32)
    colmask = (iota_l // TR
               == jax.lax.broadcasted_iota(jnp.int32, (B, B * TR), 0)
               ).astype(f32)
    inv_t = 1.0 / (T - 1)
    for kk in range(K):
        cc = jnp.clip(c_sel[kk], 0.0, 1.0)
        aw = jnp.maximum(w_sel[kk], 0.09) * (1.0 / SIGMA)
        inv2a2 = 1.0 / (2.0 * aw * aw)
        tn = jnp.floor(cc * (T - 1) + 0.5) * inv_t
        dn2 = (tn - cc) ** 2
        for r in range(8):
            tv = (jcol * 8.0 + r) * inv_t
            d = tv - cc
            w = jnp.exp((dn2 - d * d) * inv2a2) * pk[kk]
            gwbd_ref[kk * 8 + r] = w * colmask


def _k2_body(eflat_ref, dg_ref, w1_ref, b1_ref, w2_ref, b2_ref, gam_ref,
             bet_ref, gwbd_ref, out_ref, acc_ref):
    s = pl.program_id(0)
    f32 = jnp.float32

    @pl.when(s == 0)
    def _():
        acc_ref[...] = jnp.zeros((B, C), f32)

    h = jax.lax.dot_general(dg_ref[0], w1_ref[0], (((1,), (1,)), ((), ())),
                            preferred_element_type=f32) + b1_ref[0]
    h = jnp.maximum(h, 0.0)
    gw = gwbd_ref[0]
    s_vec = jax.lax.dot_general(gw, h, (((1,), (0,)), ((), ())),
                                preferred_element_type=f32)
    y = jax.lax.dot_general(s_vec, w2_ref[0], (((1,), (1,)), ((), ())),
                            preferred_element_type=f32)
    g_sum = jnp.sum(gw, axis=1, keepdims=True)
    acc_ref[...] += y + g_sum * b2_ref[0]

    @pl.when(s == NS - 1)
    def _():
        acc = acc_ref[...]
        mu = jnp.mean(acc, axis=1, keepdims=True)
        xc = acc - mu
        var = jnp.mean(xc * xc, axis=1, keepdims=True)
        out_ref[...] = xc * jax.lax.rsqrt(var + 1e-5) * gam_ref[...] \
            + bet_ref[...]


@jax.jit
def kernel(qst, data, W_in, b_in, W_out, b_out, W_router, b_router,
           W_gauss, b_gauss, W1, b1, W2, b2, gamma, beta):
    f32 = jnp.float32
    Wq, Wk, Wv = W_in[:C], W_in[C:2 * C], W_in[2 * C:]
    bq, bk, bv = (b_in[:C].reshape(1, C), b_in[C:2 * C].reshape(1, C),
                  b_in[2 * C:].reshape(1, C))
    Wg = jnp.concatenate([W_gauss[0::2], W_gauss[1::2]], axis=0)
    bg = jnp.concatenate([b_gauss[0::2], b_gauss[1::2]]).reshape(1, 2 * E)

    eflat, gwbd = pl.pallas_call(
        _k1_body,
        out_shape=[
            jax.ShapeDtypeStruct((K, B), jnp.int32),
            jax.ShapeDtypeStruct((NS, B, B * TR), f32),
        ],
    )(qst, data, Wq, Wk, Wv, bq, bk, bv, W_out, b_out.reshape(1, C),
      W_router, b_router.reshape(1, E), Wg, bg)

    data_g = data.reshape(B, TR, 8, C).transpose(2, 0, 1, 3).reshape(8, B * TR, C)
    eflat1 = eflat.reshape(NS)

    grid_spec = pltpu.PrefetchScalarGridSpec(
        num_scalar_prefetch=1,
        grid=(NS,),
        in_specs=[
            pl.BlockSpec((1, B * TR, C), lambda s, ef: (s % 8, 0, 0)),
            pl.BlockSpec((1, C // 2, C), lambda s, ef: (ef[s], 0, 0)),
            pl.BlockSpec((1, 1, C // 2), lambda s, ef: (ef[s], 0, 0)),
            pl.BlockSpec((1, C, C // 2), lambda s, ef: (ef[s], 0, 0)),
            pl.BlockSpec((1, 1, C), lambda s, ef: (ef[s], 0, 0)),
            pl.BlockSpec((1, C), lambda s, ef: (0, 0)),
            pl.BlockSpec((1, C), lambda s, ef: (0, 0)),
            pl.BlockSpec((1, B, B * TR), lambda s, ef: (s, 0, 0)),
        ],
        out_specs=pl.BlockSpec((B, C), lambda s, ef: (0, 0)),
        scratch_shapes=[pltpu.VMEM((B, C), f32)],
    )
    final = pl.pallas_call(
        _k2_body,
        grid_spec=grid_spec,
        out_shape=jax.ShapeDtypeStruct((B, C), f32),
    )(eflat1, data_g, W1, b1.reshape(E, 1, C // 2), W2,
      b2.reshape(E, 1, C), gamma.reshape(1, C), beta.reshape(1, C), gwbd)

    return final.reshape(B, 1, C)
